# Initial kernel scaffold; baseline (speedup 1.0000x reference)
import functools

import jax
import jax.numpy as jnp
from jax import lax
from jax.experimental import pallas as pl
from jax.experimental.pallas import tpu as pltpu
from jax.experimental.pallas import tpu_sc as plsc

N = 10000
E = 320000
D = 128
NC = 2
NS = 16
CHUNK = 128
CHUNKS = 80
E_PAD = NC * NS * CHUNKS * CHUNK
NP = 10240
RPT = NP // NS
NBUF = 4

_mesh = plsc.VectorSubcoreMesh(core_axis_name="c", subcore_axis_name="s")


@functools.partial(
    pl.kernel,
    out_type=jax.ShapeDtypeStruct((NC, NP), jnp.float32),
    mesh=_mesh,
    scratch_types=[
        pltpu.VMEM((CHUNKS, CHUNK), jnp.int32),
        pltpu.VMEM((RPT,), jnp.float32),
        pltpu.VMEM((CHUNK,), jnp.float32),
        pltpu.VMEM_SHARED((NP,), jnp.float32),
    ],
)
def _deg_kernel(dst_hbm, deg_out, idx_v, row_v, ones_v, acc):
    c = lax.axis_index("c")
    s = lax.axis_index("s")
    base = s * RPT
    for i in range(RPT // 16):
        row_v[pl.ds(i * 16, 16)] = jnp.zeros((16,), jnp.float32)
    for i in range(CHUNK // 16):
        ones_v[pl.ds(i * 16, 16)] = jnp.ones((16,), jnp.float32)
    pltpu.sync_copy(row_v, acc.at[pl.ds(base, RPT)])
    pltpu.sync_copy(dst_hbm.at[c, s], idx_v)
    plsc.subcore_barrier()

    def body(j, carry):
        pltpu.sync_copy(ones_v, acc.at[idx_v.at[j]], add=True)
        return carry

    lax.fori_loop(0, CHUNKS, body, 0)
    plsc.subcore_barrier()
    pltpu.sync_copy(acc.at[pl.ds(base, RPT)], row_v)
    pltpu.sync_copy(row_v, deg_out.at[c, pl.ds(base, RPT)])


@functools.partial(
    pl.kernel,
    out_type=jax.ShapeDtypeStruct((NC, NP, D), jnp.float32),
    mesh=_mesh,
    scratch_types=[
        pltpu.VMEM((CHUNKS, CHUNK), jnp.int32),
        pltpu.VMEM((CHUNKS, CHUNK), jnp.int32),
        pltpu.VMEM((CHUNK, D), jnp.float32),
        pltpu.VMEM((CHUNK, D), jnp.float32),
        pltpu.VMEM((CHUNK, D), jnp.float32),
        pltpu.VMEM((CHUNK, D), jnp.float32),
        pltpu.SemaphoreType.DMA,
        pltpu.SemaphoreType.DMA,
        pltpu.SemaphoreType.DMA,
        pltpu.SemaphoreType.DMA,
        pltpu.VMEM_SHARED((NP, D), jnp.float32),
    ],
)
def _agg_kernel(src_hbm, dst_hbm, hp_hbm, out_hbm,
                src_v, dst_v, b0, b1, b2, b3, s0, s1, s2, s3, acc):
    c = lax.axis_index("c")
    s = lax.axis_index("s")
    base = s * RPT
    bufs = (b0, b1, b2, b3)
    sems = (s0, s1, s2, s3)

    def zrow(i, carry):
        for k in range(D // 16):
            b0[i, pl.ds(k * 16, 16)] = jnp.zeros((16,), jnp.float32)
        return carry

    lax.fori_loop(0, CHUNK, zrow, 0)
    for k in range(RPT // CHUNK):
        pltpu.sync_copy(b0, acc.at[pl.ds(base + k * CHUNK, CHUNK)])
    pltpu.sync_copy(src_hbm.at[c, s], src_v)
    pltpu.sync_copy(dst_hbm.at[c, s], dst_v)
    plsc.subcore_barrier()

    for b in range(NBUF):
        pltpu.async_copy(hp_hbm.at[src_v.at[b]], bufs[b], sems[b])

    def group(g, carry):
        for b in range(NBUF):
            j = g * NBUF + b
            pltpu.make_async_copy(hp_hbm.at[pl.ds(0, CHUNK)], bufs[b],
                                  sems[b]).wait()
            pltpu.sync_copy(bufs[b], acc.at[dst_v.at[j]], add=True)
            pltpu.async_copy(hp_hbm.at[src_v.at[j + NBUF]], bufs[b], sems[b])
        return carry

    lax.fori_loop(0, CHUNKS // NBUF - 1, group, 0)
    for b in range(NBUF):
        j = (CHUNKS // NBUF - 1) * NBUF + b
        pltpu.make_async_copy(hp_hbm.at[pl.ds(0, CHUNK)], bufs[b],
                              sems[b]).wait()
        pltpu.sync_copy(bufs[b], acc.at[dst_v.at[j]], add=True)
    plsc.subcore_barrier()

    for k in range(RPT // CHUNK):
        buf = bufs[k % NBUF]
        pltpu.sync_copy(acc.at[pl.ds(base + k * CHUNK, CHUNK)], buf)
        pltpu.sync_copy(buf, out_hbm.at[c, pl.ds(base + k * CHUNK, CHUNK)])


def _xform_body(x_ref, w_ref, deg_ref, hp_ref, dinv_ref):
    deg = deg_ref[0] + deg_ref[1] + 1.0
    dinv = lax.rsqrt(deg)
    h = jnp.dot(x_ref[...], w_ref[...], preferred_element_type=jnp.float32)
    hp_ref[...] = h * dinv
    dinv_ref[...] = dinv


_BM = 512


def _xform(x_p, W, deg3):
    return pl.pallas_call(
        _xform_body,
        grid=(NP // _BM,),
        in_specs=[
            pl.BlockSpec((_BM, D), lambda i: (i, 0)),
            pl.BlockSpec((D, D), lambda i: (0, 0)),
            pl.BlockSpec((NC, _BM, 1), lambda i: (0, i, 0)),
        ],
        out_specs=[
            pl.BlockSpec((_BM, D), lambda i: (i, 0)),
            pl.BlockSpec((_BM, 1), lambda i: (i, 0)),
        ],
        out_shape=[
            jax.ShapeDtypeStruct((NP, D), jnp.float32),
            jax.ShapeDtypeStruct((NP, 1), jnp.float32),
        ],
    )(x_p, W, deg3)


def _combine_body(a0_ref, a1_ref, hp_ref, dinv_ref, b_ref, out_ref):
    out_ref[...] = (dinv_ref[...] * (a0_ref[...] + a1_ref[...] + hp_ref[...])
                    + b_ref[...])


def _combine(a0, a1, hp, dinv, b2):
    return pl.pallas_call(
        _combine_body,
        grid=(NP // _BM,),
        in_specs=[
            pl.BlockSpec((_BM, D), lambda i: (i, 0)),
            pl.BlockSpec((_BM, D), lambda i: (i, 0)),
            pl.BlockSpec((_BM, D), lambda i: (i, 0)),
            pl.BlockSpec((_BM, 1), lambda i: (i, 0)),
            pl.BlockSpec((1, D), lambda i: (0, 0)),
        ],
        out_specs=pl.BlockSpec((_BM, D), lambda i: (i, 0)),
        out_shape=jax.ShapeDtypeStruct((NP, D), jnp.float32),
    )(a0, a1, hp, dinv, b2)


def kernel(x, edge_index, W, b):
    src = edge_index[0].astype(jnp.int32)
    dst = edge_index[1].astype(jnp.int32)
    pad = jnp.arange(E_PAD - E, dtype=jnp.int32) % (NP - N) + N
    src_p = jnp.concatenate([src, pad]).reshape(NC, NS, CHUNKS, CHUNK)
    dst_p = jnp.concatenate([dst, pad]).reshape(NC, NS, CHUNKS, CHUNK)
    x_p = jnp.pad(x, ((0, NP - N), (0, 0)))

    deg = _deg_kernel(dst_p)
    hp, dinv = _xform(x_p, W, deg[..., None])
    acc = _agg_kernel(src_p, dst_p, hp)
    out = _combine(acc[0], acc[1], hp, dinv, b.reshape(1, D))
    return out[:N]


# speedup vs baseline: 40.1669x; 40.1669x over previous
import functools

import jax
import jax.numpy as jnp
from jax import lax
from jax.experimental import pallas as pl
from jax.experimental.pallas import tpu as pltpu
from jax.experimental.pallas import tpu_sc as plsc

N = 10000
E = 320000
D = 128
DH = 64
NC = 2
NS = 16
NW = NC * NS
CHUNK = 128
NCHUNKS = E // CHUNK
CPW = NCHUNKS // NW
NEXTRA = NCHUNKS - CPW * NW
NP = 10240
RPT = NP // NS
NBUF = 6

_mesh = plsc.VectorSubcoreMesh(core_axis_name="c", subcore_axis_name="s")


@functools.partial(
    pl.kernel,
    out_type=jax.ShapeDtypeStruct((NC, NP), jnp.float32),
    mesh=_mesh,
    compiler_params=pltpu.CompilerParams(use_tc_tiling_on_sc=False),
    scratch_types=[
        pltpu.VMEM((CPW + 1, CHUNK), jnp.int32),
        pltpu.VMEM((RPT,), jnp.float32),
        pltpu.VMEM((CHUNK,), jnp.float32),
        pltpu.VMEM_SHARED((NP,), jnp.float32),
    ],
)
def _deg_kernel(e_hbm, deg_out, idx_v, row_v, ones_v, acc):
    c = lax.axis_index("c")
    s = lax.axis_index("s")
    wid = c * NS + s
    base = s * RPT
    for i in range(RPT // 16):
        row_v[pl.ds(i * 16, 16)] = jnp.zeros((16,), jnp.float32)
    for i in range(CHUNK // 16):
        ones_v[pl.ds(i * 16, 16)] = jnp.ones((16,), jnp.float32)
    pltpu.sync_copy(row_v, acc.at[pl.ds(base, RPT)])
    pltpu.sync_copy(e_hbm.at[1, pl.ds(wid * CPW, CPW)],
                    idx_v.at[pl.ds(0, CPW)])

    @pl.when(wid < NEXTRA)
    def _():
        pltpu.sync_copy(e_hbm.at[1, pl.ds(NW * CPW + wid, 1)],
                        idx_v.at[pl.ds(CPW, 1)])

    plsc.subcore_barrier()

    def body(j, carry):
        pltpu.sync_copy(ones_v, acc.at[idx_v.at[j]], add=True)
        return carry

    lax.fori_loop(0, CPW, body, 0)

    @pl.when(wid < NEXTRA)
    def _():
        pltpu.sync_copy(ones_v, acc.at[idx_v.at[CPW]], add=True)

    plsc.subcore_barrier()
    pltpu.sync_copy(acc.at[pl.ds(base, RPT)], row_v)
    pltpu.sync_copy(row_v, deg_out.at[c, pl.ds(base, RPT)])


_RPW = NP // NW


@functools.partial(
    pl.kernel,
    out_type=jax.ShapeDtypeStruct((NP,), jnp.float32),
    mesh=_mesh,
    compiler_params=pltpu.CompilerParams(use_tc_tiling_on_sc=False,
                                         needs_layout_passes=False),
    scratch_types=[
        pltpu.VMEM((_RPW,), jnp.float32),
        pltpu.VMEM((_RPW,), jnp.float32),
    ],
)
def _dinv_kernel(deg_hbm, dinv_out, v0, v1):
    c = lax.axis_index("c")
    s = lax.axis_index("s")
    base = (s * NC + c) * _RPW
    pltpu.sync_copy(deg_hbm.at[0, pl.ds(base, _RPW)], v0)
    pltpu.sync_copy(deg_hbm.at[1, pl.ds(base, _RPW)], v1)
    for i in range(_RPW // 16):
        d = v0[pl.ds(i * 16, 16)] + v1[pl.ds(i * 16, 16)] + 1.0
        y = plsc.bitcast(
            jnp.int32(0x5F3759DF) - (plsc.bitcast(d, jnp.int32) >> 1),
            jnp.float32)
        h = -0.5 * d
        y = y * (1.5 + h * y * y)
        y = y * (1.5 + h * y * y)
        y = y * (1.5 + h * y * y)
        v0[pl.ds(i * 16, 16)] = y
    pltpu.sync_copy(v0, dinv_out.at[pl.ds(base, _RPW)])


@functools.partial(
    pl.kernel,
    out_type=[jax.ShapeDtypeStruct((NP, DH), jnp.float32),
              jax.ShapeDtypeStruct((NP, DH), jnp.float32)],
    mesh=_mesh,
    compiler_params=pltpu.CompilerParams(use_tc_tiling_on_sc=False),
    scratch_types=[
        pltpu.VMEM((CPW + 1, CHUNK), jnp.int32),
        pltpu.VMEM((CPW + 1, CHUNK), jnp.int32),
        [pltpu.VMEM((CHUNK, DH), jnp.float32)] * NBUF,
        [pltpu.SemaphoreType.DMA] * NBUF,
        pltpu.VMEM_SHARED((NP, DH), jnp.float32),
    ],
)
def _agg_kernel(e_hbm, hp_hbm, out0_hbm, out1_hbm,
                src_v, dst_v, bufs, gsems, acc):
    c = lax.axis_index("c")
    s = lax.axis_index("s")
    wid = c * NS + s
    base = s * RPT

    def zrow(i, carry):
        for k in range(DH // 16):
            bufs[0][i, pl.ds(k * 16, 16)] = jnp.zeros((16,), jnp.float32)
        return carry

    lax.fori_loop(0, CHUNK, zrow, 0)
    for k in range(RPT // CHUNK):
        pltpu.sync_copy(bufs[0], acc.at[pl.ds(base + k * CHUNK, CHUNK)])
    pltpu.sync_copy(e_hbm.at[0, pl.ds(wid * CPW, CPW)],
                    src_v.at[pl.ds(0, CPW)])
    pltpu.sync_copy(e_hbm.at[1, pl.ds(wid * CPW, CPW)],
                    dst_v.at[pl.ds(0, CPW)])

    @pl.when(wid < NEXTRA)
    def _():
        pltpu.sync_copy(e_hbm.at[0, pl.ds(NW * CPW + wid, 1)],
                        src_v.at[pl.ds(CPW, 1)])
        pltpu.sync_copy(e_hbm.at[1, pl.ds(NW * CPW + wid, 1)],
                        dst_v.at[pl.ds(CPW, 1)])

    plsc.subcore_barrier()

    for j0 in range(NBUF):
        pltpu.async_copy(hp_hbm.at[src_v.at[j0]], bufs[j0], gsems[j0])

    def group(g, carry):
        for u in range(NBUF):
            j = g * NBUF + u
            pltpu.make_async_copy(hp_hbm.at[pl.ds(0, CHUNK)], bufs[u],
                                  gsems[u]).wait()
            pltpu.sync_copy(bufs[u], acc.at[dst_v.at[j]], add=True)
            pltpu.async_copy(hp_hbm.at[src_v.at[j + NBUF]], bufs[u],
                             gsems[u])
        return carry

    lax.fori_loop(0, CPW // NBUF - 1, group, 0)
    for u in range(NBUF):
        j = (CPW // NBUF - 1) * NBUF + u
        pltpu.make_async_copy(hp_hbm.at[pl.ds(0, CHUNK)], bufs[u],
                              gsems[u]).wait()
        pltpu.sync_copy(bufs[u], acc.at[dst_v.at[j]], add=True)

    @pl.when(wid < NEXTRA)
    def _():
        pltpu.async_copy(hp_hbm.at[src_v.at[CPW]], bufs[0], gsems[0])
        pltpu.make_async_copy(hp_hbm.at[pl.ds(0, CHUNK)], bufs[0],
                              gsems[0]).wait()
        pltpu.sync_copy(bufs[0], acc.at[dst_v.at[CPW]], add=True)

    plsc.subcore_barrier()

    for k in range(RPT // CHUNK):
        buf = bufs[k % NBUF]
        pltpu.sync_copy(acc.at[pl.ds(base + k * CHUNK, CHUNK)], buf)

        @pl.when(c == 0)
        def _():
            pltpu.sync_copy(buf, out0_hbm.at[pl.ds(base + k * CHUNK, CHUNK)])

        @pl.when(c == 1)
        def _():
            pltpu.sync_copy(buf, out1_hbm.at[pl.ds(base + k * CHUNK, CHUNK)])


def _xform_body(x_ref, w_ref, dinv_ref, hp_ref):
    h = jnp.dot(x_ref[...], w_ref[...], preferred_element_type=jnp.float32)
    hp_ref[...] = h * dinv_ref[...]


_BM = 1000


def _xform(x, W, dinv):
    return pl.pallas_call(
        _xform_body,
        grid=(N // _BM,),
        in_specs=[
            pl.BlockSpec((_BM, D), lambda i: (i, 0)),
            pl.BlockSpec((D, D), lambda i: (0, 0)),
            pl.BlockSpec((_BM, 1), lambda i: (i, 0)),
        ],
        out_specs=pl.BlockSpec((_BM, D), lambda i: (i, 0)),
        out_shape=jax.ShapeDtypeStruct((N, D), jnp.float32),
    )(x, W, dinv)


def _combine_body(aa0_ref, aa1_ref, ab0_ref, ab1_ref, hp_ref, dinv_ref,
                  b_ref, out_ref):
    dinv = dinv_ref[...]
    bias = b_ref[...]
    hp = hp_ref[...]
    outa = dinv * (aa0_ref[...] + aa1_ref[...] + hp[:, :DH]) + bias[:, :DH]
    outb = dinv * (ab0_ref[...] + ab1_ref[...] + hp[:, DH:]) + bias[:, DH:]
    out_ref[...] = jnp.concatenate([outa, outb], axis=1)


def _combine(aa0, aa1, ab0, ab1, hp, dinv, b2):
    half = pl.BlockSpec((_BM, DH), lambda i: (i, 0))
    return pl.pallas_call(
        _combine_body,
        grid=(N // _BM,),
        in_specs=[
            half, half, half, half,
            pl.BlockSpec((_BM, D), lambda i: (i, 0)),
            pl.BlockSpec((_BM, 1), lambda i: (i, 0)),
            pl.BlockSpec((1, D), lambda i: (0, 0)),
        ],
        out_specs=pl.BlockSpec((_BM, D), lambda i: (i, 0)),
        out_shape=jax.ShapeDtypeStruct((N, D), jnp.float32),
    )(aa0, aa1, ab0, ab1, hp, dinv, b2)


def kernel(x, edge_index, W, b):
    e = edge_index.astype(jnp.int32).reshape(2, NCHUNKS, CHUNK)
    deg = _deg_kernel(e)
    dinv = _dinv_kernel(deg)[:, None]
    hp = _xform(x, W, dinv)
    aa0, aa1 = _agg_kernel(e, hp[:, :DH])
    ab0, ab1 = _agg_kernel(e, hp[:, DH:])
    return _combine(aa0, aa1, ab0, ab1, hp, dinv, b.reshape(1, D))


# speedup vs baseline: 41.2781x; 1.0277x over previous
import functools

import jax
import jax.numpy as jnp
from jax import lax
from jax.experimental import pallas as pl
from jax.experimental.pallas import tpu as pltpu
from jax.experimental.pallas import tpu_sc as plsc

N = 10000
E = 320000
D = 128
DH = 64
NC = 2
NS = 16
NW = NC * NS
CHUNK = 128
NCHUNKS = E // CHUNK
CPW = NCHUNKS // NW
NEXTRA = NCHUNKS - CPW * NW
NP = 10240
RPT = NP // NS
NBUF = 6

_mesh = plsc.VectorSubcoreMesh(core_axis_name="c", subcore_axis_name="s")


@functools.partial(
    pl.kernel,
    out_type=jax.ShapeDtypeStruct((NC, NP), jnp.float32),
    mesh=_mesh,
    compiler_params=pltpu.CompilerParams(use_tc_tiling_on_sc=False),
    scratch_types=[
        pltpu.VMEM((CPW + 1, CHUNK), jnp.int32),
        pltpu.VMEM((RPT,), jnp.float32),
        pltpu.VMEM((CHUNK,), jnp.float32),
        [pltpu.SemaphoreType.DMA] * NBUF,
        pltpu.VMEM_SHARED((NP,), jnp.float32),
    ],
)
def _deg_kernel(e_hbm, deg_out, idx_v, row_v, ones_v, dsems, acc):
    c = lax.axis_index("c")
    s = lax.axis_index("s")
    wid = c * NS + s
    base = s * RPT
    for i in range(RPT // 16):
        row_v[pl.ds(i * 16, 16)] = jnp.zeros((16,), jnp.float32)
    for i in range(CHUNK // 16):
        ones_v[pl.ds(i * 16, 16)] = jnp.ones((16,), jnp.float32)
    pltpu.sync_copy(row_v, acc.at[pl.ds(base, RPT)])
    pltpu.sync_copy(e_hbm.at[1, pl.ds(wid * CPW, CPW)],
                    idx_v.at[pl.ds(0, CPW)])

    @pl.when(wid < NEXTRA)
    def _():
        pltpu.sync_copy(e_hbm.at[1, pl.ds(NW * CPW + wid, 1)],
                        idx_v.at[pl.ds(CPW, 1)])

    plsc.subcore_barrier()

    def body(g, carry):
        for u in range(NBUF):
            @pl.when(g > 0)
            def _():
                pltpu.make_async_copy(ones_v, acc.at[pl.ds(0, CHUNK)],
                                      dsems[u]).wait()
            pltpu.async_copy(ones_v, acc.at[idx_v.at[g * NBUF + u]],
                             dsems[u], add=True)
        return carry

    lax.fori_loop(0, CPW // NBUF, body, 0)
    for u in range(NBUF):
        pltpu.make_async_copy(ones_v, acc.at[pl.ds(0, CHUNK)],
                              dsems[u]).wait()

    @pl.when(wid < NEXTRA)
    def _():
        pltpu.sync_copy(ones_v, acc.at[idx_v.at[CPW]], add=True)

    plsc.subcore_barrier()
    pltpu.sync_copy(acc.at[pl.ds(base, RPT)], row_v)
    pltpu.sync_copy(row_v, deg_out.at[c, pl.ds(base, RPT)])


_RPW = NP // NW


@functools.partial(
    pl.kernel,
    out_type=jax.ShapeDtypeStruct((NP,), jnp.float32),
    mesh=_mesh,
    compiler_params=pltpu.CompilerParams(use_tc_tiling_on_sc=False,
                                         needs_layout_passes=False),
    scratch_types=[
        pltpu.VMEM((_RPW,), jnp.float32),
        pltpu.VMEM((_RPW,), jnp.float32),
    ],
)
def _dinv_kernel(deg_hbm, dinv_out, v0, v1):
    c = lax.axis_index("c")
    s = lax.axis_index("s")
    base = (s * NC + c) * _RPW
    pltpu.sync_copy(deg_hbm.at[0, pl.ds(base, _RPW)], v0)
    pltpu.sync_copy(deg_hbm.at[1, pl.ds(base, _RPW)], v1)
    for i in range(_RPW // 16):
        d = v0[pl.ds(i * 16, 16)] + v1[pl.ds(i * 16, 16)] + 1.0
        y = plsc.bitcast(
            jnp.int32(0x5F3759DF) - (plsc.bitcast(d, jnp.int32) >> 1),
            jnp.float32)
        h = -0.5 * d
        y = y * (1.5 + h * y * y)
        y = y * (1.5 + h * y * y)
        y = y * (1.5 + h * y * y)
        v0[pl.ds(i * 16, 16)] = y
    pltpu.sync_copy(v0, dinv_out.at[pl.ds(base, _RPW)])


@functools.partial(
    pl.kernel,
    out_type=[jax.ShapeDtypeStruct((NP, DH), jnp.float32),
              jax.ShapeDtypeStruct((NP, DH), jnp.float32)],
    mesh=_mesh,
    compiler_params=pltpu.CompilerParams(use_tc_tiling_on_sc=False),
    scratch_types=[
        pltpu.VMEM((CPW + 1, CHUNK), jnp.int32),
        pltpu.VMEM((CPW + 1, CHUNK), jnp.int32),
        [pltpu.VMEM((CHUNK, DH), jnp.float32)] * NBUF,
        [pltpu.SemaphoreType.DMA] * NBUF,
        pltpu.VMEM_SHARED((NP, DH), jnp.float32),
    ],
)
def _agg_kernel(e_hbm, hp_hbm, out0_hbm, out1_hbm,
                src_v, dst_v, bufs, gsems, acc):
    c = lax.axis_index("c")
    s = lax.axis_index("s")
    wid = c * NS + s
    base = s * RPT

    def zrow(i, carry):
        for k in range(DH // 16):
            bufs[0][i, pl.ds(k * 16, 16)] = jnp.zeros((16,), jnp.float32)
        return carry

    lax.fori_loop(0, CHUNK, zrow, 0)
    for k in range(RPT // CHUNK):
        pltpu.sync_copy(bufs[0], acc.at[pl.ds(base + k * CHUNK, CHUNK)])
    pltpu.sync_copy(e_hbm.at[0, pl.ds(wid * CPW, CPW)],
                    src_v.at[pl.ds(0, CPW)])
    pltpu.sync_copy(e_hbm.at[1, pl.ds(wid * CPW, CPW)],
                    dst_v.at[pl.ds(0, CPW)])

    @pl.when(wid < NEXTRA)
    def _():
        pltpu.sync_copy(e_hbm.at[0, pl.ds(NW * CPW + wid, 1)],
                        src_v.at[pl.ds(CPW, 1)])
        pltpu.sync_copy(e_hbm.at[1, pl.ds(NW * CPW + wid, 1)],
                        dst_v.at[pl.ds(CPW, 1)])

    plsc.subcore_barrier()

    for j0 in range(NBUF):
        pltpu.async_copy(hp_hbm.at[src_v.at[j0]], bufs[j0], gsems[j0])

    def group(g, carry):
        for u in range(NBUF):
            j = g * NBUF + u
            pltpu.make_async_copy(hp_hbm.at[pl.ds(0, CHUNK)], bufs[u],
                                  gsems[u]).wait()
            pltpu.sync_copy(bufs[u], acc.at[dst_v.at[j]], add=True)
            pltpu.async_copy(hp_hbm.at[src_v.at[j + NBUF]], bufs[u],
                             gsems[u])
        return carry

    lax.fori_loop(0, CPW // NBUF - 1, group, 0)
    for u in range(NBUF):
        j = (CPW // NBUF - 1) * NBUF + u
        pltpu.make_async_copy(hp_hbm.at[pl.ds(0, CHUNK)], bufs[u],
                              gsems[u]).wait()
        pltpu.sync_copy(bufs[u], acc.at[dst_v.at[j]], add=True)

    @pl.when(wid < NEXTRA)
    def _():
        pltpu.async_copy(hp_hbm.at[src_v.at[CPW]], bufs[0], gsems[0])
        pltpu.make_async_copy(hp_hbm.at[pl.ds(0, CHUNK)], bufs[0],
                              gsems[0]).wait()
        pltpu.sync_copy(bufs[0], acc.at[dst_v.at[CPW]], add=True)

    plsc.subcore_barrier()

    for k in range(RPT // CHUNK):
        buf = bufs[k % NBUF]
        pltpu.sync_copy(acc.at[pl.ds(base + k * CHUNK, CHUNK)], buf)

        @pl.when(c == 0)
        def _():
            pltpu.sync_copy(buf, out0_hbm.at[pl.ds(base + k * CHUNK, CHUNK)])

        @pl.when(c == 1)
        def _():
            pltpu.sync_copy(buf, out1_hbm.at[pl.ds(base + k * CHUNK, CHUNK)])


def _xform_body(x_ref, w_ref, dinv_ref, hp_ref):
    h = jnp.dot(x_ref[...], w_ref[...], preferred_element_type=jnp.float32)
    hp_ref[...] = h * dinv_ref[...]


_BM = 1000


def _xform(x, W, dinv):
    return pl.pallas_call(
        _xform_body,
        grid=(N // _BM,),
        in_specs=[
            pl.BlockSpec((_BM, D), lambda i: (i, 0)),
            pl.BlockSpec((D, D), lambda i: (0, 0)),
            pl.BlockSpec((_BM, 1), lambda i: (i, 0)),
        ],
        out_specs=pl.BlockSpec((_BM, D), lambda i: (i, 0)),
        out_shape=jax.ShapeDtypeStruct((N, D), jnp.float32),
    )(x, W, dinv)


def _combine_body(aa0_ref, aa1_ref, ab0_ref, ab1_ref, hp_ref, dinv_ref,
                  b_ref, out_ref):
    dinv = dinv_ref[...]
    bias = b_ref[...]
    hp = hp_ref[...]
    outa = dinv * (aa0_ref[...] + aa1_ref[...] + hp[:, :DH]) + bias[:, :DH]
    outb = dinv * (ab0_ref[...] + ab1_ref[...] + hp[:, DH:]) + bias[:, DH:]
    out_ref[...] = jnp.concatenate([outa, outb], axis=1)


def _combine(aa0, aa1, ab0, ab1, hp, dinv, b2):
    half = pl.BlockSpec((_BM, DH), lambda i: (i, 0))
    return pl.pallas_call(
        _combine_body,
        grid=(N // _BM,),
        in_specs=[
            half, half, half, half,
            pl.BlockSpec((_BM, D), lambda i: (i, 0)),
            pl.BlockSpec((_BM, 1), lambda i: (i, 0)),
            pl.BlockSpec((1, D), lambda i: (0, 0)),
        ],
        out_specs=pl.BlockSpec((_BM, D), lambda i: (i, 0)),
        out_shape=jax.ShapeDtypeStruct((N, D), jnp.float32),
    )(aa0, aa1, ab0, ab1, hp, dinv, b2)


def kernel(x, edge_index, W, b):
    e = edge_index.astype(jnp.int32).reshape(2, NCHUNKS, CHUNK)
    deg = _deg_kernel(e)
    dinv = _dinv_kernel(deg)[:, None]
    hp = _xform(x, W, dinv)
    aa0, aa1 = _agg_kernel(e, hp[:, :DH])
    ab0, ab1 = _agg_kernel(e, hp[:, DH:])
    return _combine(aa0, aa1, ab0, ab1, hp, dinv, b.reshape(1, D))
